# Initial kernel scaffold; baseline (speedup 1.0000x reference)
import jax
import jax.numpy as jnp
from jax.experimental import pallas as pl


def kernel(x):
    raise NotImplementedError("write your pallas kernel here")


# speedup vs baseline: 2553.3777x; 2553.3777x over previous
import jax
import jax.numpy as jnp
from jax.experimental import pallas as pl
from jax.experimental.pallas import tpu as pltpu

_STEPS = 32
_H = 224
_W = 224


def _ecc_kernel(ts_ref, x_ref, o_ref):
    x = x_ref[0]
    h, w = x.shape
    inf = jnp.float32(jnp.inf)
    inf_col = jnp.full((h, 1), inf, jnp.float32)
    inf_row = jnp.full((1, w), inf, jnp.float32)
    xr = jnp.concatenate([x[:, 1:], inf_col], axis=1)
    hmax = jnp.maximum(x, xr)
    xd = jnp.concatenate([x[1:, :], inf_row], axis=0)
    vmax = jnp.maximum(x, xd)
    hmax_d = jnp.concatenate([hmax[1:, :], inf_row], axis=0)
    smax = jnp.maximum(hmax, hmax_d)

    one = jnp.float32(1.0)
    zero = jnp.float32(0.0)
    parts = []
    for k in range(_STEPS):
        t = ts_ref[0, k]
        pos = jnp.where(x <= t, one, zero) + jnp.where(smax <= t, one, zero)
        neg = jnp.where(hmax <= t, one, zero) + jnp.where(vmax <= t, one, zero)
        parts.append(jnp.sum(pos - neg, axis=0, keepdims=True))
    s = jnp.concatenate(parts, axis=0)
    o_ref[0] = jnp.sum(s, axis=1, keepdims=True)


def _ecc(x, *, interpret=False):
    b, c, h, w = x.shape
    n = b * c
    xs = x.reshape(n, h, w)
    ts = jnp.linspace(0.0, 1.0, _STEPS).astype(jnp.float32).reshape(1, _STEPS)
    out = pl.pallas_call(
        _ecc_kernel,
        grid=(n,),
        in_specs=[
            pl.BlockSpec(memory_space=pltpu.SMEM),
            pl.BlockSpec((1, h, w), lambda i: (i, 0, 0)),
        ],
        out_specs=pl.BlockSpec((1, _STEPS, 1), lambda i: (i, 0, 0)),
        out_shape=jax.ShapeDtypeStruct((n, _STEPS, 1), jnp.float32),
        compiler_params=pltpu.CompilerParams(
            dimension_semantics=("parallel",),
        ),
        name="cub_ecc",
        interpret=interpret,
    )(ts, xs)
    return out.reshape(b, c * _STEPS)


def kernel(x):
    return _ecc(x)
